# baseline (speedup 1.0000x reference)
import jax
import jax.numpy as jnp
from jax.experimental import pallas as pl
from jax.experimental.pallas import tpu as pltpu

DEPTH = 11
IN_W = 1024
HID_W = 8
OUT_W = 1024
N_LEAVES = 2 ** DEPTH
N_NODES = 2 ** DEPTH - 1
TILE_L = 128
N_TILES = N_LEAVES // TILE_L
B = 8


def _up_matrix(w: int, r: int):
    row = jax.lax.broadcasted_iota(jnp.int32, (w, w * r), 0)
    col = jax.lax.broadcasted_iota(jnp.int32, (w, w * r), 1)
    return (row == col // r).astype(jnp.float32)


def _fff_kernel(x_ref, xe_ref, nw_ref, nb_ref, w1_ref, b1q_ref, w2_ref, b2_ref,
                out_ref, mix_ref):
    t = pl.program_id(0)

    @pl.when(t == 0)
    def _init():
        x = x_ref[...]
        logits = jax.lax.dot_general(
            x, nw_ref[...], (((1,), (1,)), ((), ())),
            preferred_element_type=jnp.float32,
            precision=jax.lax.Precision.HIGHEST)
        logits = logits + nb_ref[...]
        s = jax.nn.sigmoid(logits)
        m = jnp.concatenate([1.0 - s[:, 0:1], s[:, 0:1]], axis=1)
        for d in range(1, DEPTH):
            n = 2 ** d
            sd = s[:, n - 1:2 * n - 1]
            U = _up_matrix(n, 2)
            u = jnp.dot(m, U, preferred_element_type=jnp.float32,
                        precision=jax.lax.Precision.HIGHEST)
            us = jnp.dot(sd, U, preferred_element_type=jnp.float32,
                         precision=jax.lax.Precision.HIGHEST)
            par = (jax.lax.broadcasted_iota(jnp.int32, (B, 2 * n), 1) & 1
                   ).astype(jnp.float32)
            mod = (1.0 - par) + us * (2.0 * par - 1.0)
            m = u * mod
        for tt in range(N_TILES):
            mix_ref[tt] = m[:, tt * TILE_L:(tt + 1) * TILE_L]
        out_ref[...] = jnp.zeros((B, OUT_W), jnp.float32)

    h = None
    for r in range(64):
        p = jax.lax.dot_general(w1_ref[:, r, :], xe_ref[r],
                                (((1,), (0,)), ((), ())),
                                preferred_element_type=jnp.float32)
        h = p if h is None else h + p
    hq = jnp.transpose(h)
    ms = mix_ref[t]
    mq = jnp.concatenate([ms] * HID_W, axis=0)
    gq = jnp.maximum(hq + b1q_ref[0], 0.0) * mq
    acc = jax.lax.dot_general(ms, b2_ref[...], (((1,), (0,)), ((), ())),
                              preferred_element_type=jnp.float32)
    for j in range(HID_W):
        acc = acc + jax.lax.dot_general(
            gq[HID_W * j:HID_W * (j + 1), :], w2_ref[:, j, :],
            (((1,), (0,)), ((), ())), preferred_element_type=jnp.float32)
    out_ref[...] += acc


def kernel(x, node_weights, node_biases, w1s, b1s, w2s, b2s):
    orig_shape = x.shape
    x2 = x.reshape(-1, x.shape[-1])
    nb_row = node_biases.reshape(1, N_NODES)
    e8 = jnp.eye(HID_W, dtype=x2.dtype)
    x3 = x2.T.reshape(64, 16, HID_W)
    xr = (e8[None, None, :, :, None] * x3[:, :, None, None, :]
          ).reshape(64, 128, HID_W * B)
    w1v = w1s.reshape(N_LEAVES, 64, 128)
    b1r = b1s.reshape(N_TILES, TILE_L, HID_W).transpose(0, 2, 1)
    b1q = jnp.broadcast_to(b1r[:, :, None, :],
                           (N_TILES, HID_W, B, TILE_L)
                           ).reshape(N_TILES, HID_W * B, TILE_L)
    out = pl.pallas_call(
        _fff_kernel,
        grid=(N_TILES,),
        in_specs=[
            pl.BlockSpec((B, IN_W), lambda t: (0, 0)),
            pl.BlockSpec((64, 128, HID_W * B), lambda t: (0, 0, 0)),
            pl.BlockSpec((N_NODES, IN_W), lambda t: (0, 0)),
            pl.BlockSpec((1, N_NODES), lambda t: (0, 0)),
            pl.BlockSpec((TILE_L, 64, 128), lambda t: (t, 0, 0)),
            pl.BlockSpec((1, HID_W * B, TILE_L), lambda t: (t, 0, 0)),
            pl.BlockSpec((TILE_L, HID_W, OUT_W), lambda t: (t, 0, 0)),
            pl.BlockSpec((TILE_L, OUT_W), lambda t: (t, 0)),
        ],
        out_specs=pl.BlockSpec((B, OUT_W), lambda t: (0, 0)),
        out_shape=jax.ShapeDtypeStruct((B, OUT_W), jnp.float32),
        scratch_shapes=[
            pltpu.VMEM((N_TILES, B, TILE_L), jnp.float32),
        ],
        compiler_params=pltpu.CompilerParams(
            dimension_semantics=("arbitrary",),
        ),
    )(x2, xr, node_weights, nb_row, w1v, b1q, w2s, b2s)
    return out.reshape(*orig_shape[:-1], OUT_W)


# speedup vs baseline: 1.1059x; 1.1059x over previous
import jax
import jax.numpy as jnp
from jax.experimental import pallas as pl
from jax.experimental.pallas import tpu as pltpu

DEPTH = 11
IN_W = 1024
HID_W = 8
OUT_W = 1024
N_LEAVES = 2 ** DEPTH
N_NODES = 2 ** DEPTH - 1
TILE_L = 64
N_TILES = N_LEAVES // TILE_L
B = 8

_HI = jax.lax.Precision.HIGHEST


def _up_matrix(w: int, r: int):
    row = jax.lax.broadcasted_iota(jnp.int32, (w, w * r), 0)
    col = jax.lax.broadcasted_iota(jnp.int32, (w, w * r), 1)
    return (row == col // r).astype(jnp.float32)


def _shift_add(z, shifts):
    for s in shifts:
        z = z + jnp.concatenate([z[:, s:], z[:, :s]], axis=1)
    return z


def _fff_kernel(x_ref, v_ref, nw_ref, nb_ref, w1_ref, b1t_ref, w2_ref, b2_ref,
                out_ref, mix_ref, mask_ref, f_ref, ys_ref, gs_ref):
    t = pl.program_id(0)

    @pl.when(t == 0)
    def _init():
        x = x_ref[...]
        logits = jax.lax.dot_general(
            x, nw_ref[...], (((1,), (1,)), ((), ())),
            preferred_element_type=jnp.float32, precision=_HI)
        logits = logits + nb_ref[...]
        s = jax.nn.sigmoid(logits)
        m = jnp.concatenate([1.0 - s[:, 0:1], s[:, 0:1]], axis=1)
        for d in range(1, DEPTH):
            n = 2 ** d
            sd = s[:, n - 1:2 * n - 1]
            U = _up_matrix(n, 2)
            u = jnp.dot(m, U, preferred_element_type=jnp.float32, precision=_HI)
            us = jnp.dot(sd, U, preferred_element_type=jnp.float32,
                         precision=_HI)
            par = (jax.lax.broadcasted_iota(jnp.int32, (B, 2 * n), 1) & 1
                   ).astype(jnp.float32)
            mod = (1.0 - par) + us * (2.0 * par - 1.0)
            m = u * mod
        eyeb = (jax.lax.broadcasted_iota(jnp.int32, (B, B), 0) ==
                jax.lax.broadcasted_iota(jnp.int32, (B, B), 1)
                ).astype(jnp.float32)
        mt = jax.lax.dot_general(m, eyeb, (((0,), (0,)), ((), ())),
                                 preferred_element_type=jnp.float32,
                                 precision=_HI)
        for tt in range(N_TILES):
            mix_ref[tt] = mt[tt * TILE_L:(tt + 1) * TILE_L, :]
        ci = jax.lax.broadcasted_iota(jnp.int32, (128, 128), 0)
        c2i = jax.lax.broadcasted_iota(jnp.int32, (128, 128), 1)
        mask_ref[...] = ((ci // 8) == (c2i // 8)).astype(jnp.bfloat16)
        ji = jax.lax.broadcasted_iota(jnp.int32, (HID_W, 128), 0)
        jc = jax.lax.broadcasted_iota(jnp.int32, (HID_W, 128), 1)
        f_ref[...] = ((jc % 8) == ji).astype(jnp.bfloat16)
        out_ref[...] = jnp.zeros((B, OUT_W), jnp.float32)

    v = v_ref[...]
    mask = mask_ref[...]
    mslab = mix_ref[t]
    for l in range(TILE_L):
        y = jax.lax.dot_general(w1_ref[l].astype(jnp.bfloat16), v,
                                (((0,), (0,)), ((), ())),
                                preferred_element_type=jnp.float32)
        ys_ref[:, 128 * l:128 * (l + 1)] = y.astype(jnp.bfloat16) * mask
    z = jax.lax.dot_general(f_ref[...], ys_ref[...], (((1,), (0,)), ((), ())),
                            preferred_element_type=jnp.float32)
    z = _shift_add(z, (8, 16, 32, 64))
    b1t = b1t_ref[0]
    for l in range(TILE_L):
        ht = z[:, 128 * l:128 * l + 8] + b1t[:, l:l + 1]
        gs_ref[8 * l:8 * (l + 1), :] = jnp.maximum(ht, 0.0) * mslab[l:l + 1, :]
    w2f = w2_ref[...].reshape(TILE_L * HID_W, OUT_W)
    acc = jax.lax.dot_general(gs_ref[...], w2f, (((0,), (0,)), ((), ())),
                              preferred_element_type=jnp.float32)
    acc = acc + jax.lax.dot_general(mslab, b2_ref[...],
                                    (((0,), (0,)), ((), ())),
                                    preferred_element_type=jnp.float32)
    out_ref[...] += acc


def kernel(x, node_weights, node_biases, w1s, b1s, w2s, b2s):
    orig_shape = x.shape
    x2 = x.reshape(-1, x.shape[-1])
    nb_row = node_biases.reshape(1, N_NODES)
    v = x2.reshape(B, 64, 16).transpose(1, 2, 0).reshape(64, 128)
    v = v.astype(jnp.bfloat16)
    w1d = w1s.reshape(N_LEAVES, 64, 128)
    b1t = b1s.reshape(N_TILES, TILE_L, HID_W).transpose(0, 2, 1)
    out = pl.pallas_call(
        _fff_kernel,
        grid=(N_TILES,),
        in_specs=[
            pl.BlockSpec((B, IN_W), lambda t: (0, 0)),
            pl.BlockSpec((64, 128), lambda t: (0, 0)),
            pl.BlockSpec((N_NODES, IN_W), lambda t: (0, 0)),
            pl.BlockSpec((1, N_NODES), lambda t: (0, 0)),
            pl.BlockSpec((TILE_L, 64, 128), lambda t: (t, 0, 0)),
            pl.BlockSpec((1, HID_W, TILE_L), lambda t: (t, 0, 0)),
            pl.BlockSpec((TILE_L, HID_W, OUT_W), lambda t: (t, 0, 0)),
            pl.BlockSpec((TILE_L, OUT_W), lambda t: (t, 0)),
        ],
        out_specs=pl.BlockSpec((B, OUT_W), lambda t: (0, 0)),
        out_shape=jax.ShapeDtypeStruct((B, OUT_W), jnp.float32),
        scratch_shapes=[
            pltpu.VMEM((N_TILES, TILE_L, B), jnp.float32),
            pltpu.VMEM((128, 128), jnp.bfloat16),
            pltpu.VMEM((HID_W, 128), jnp.bfloat16),
            pltpu.VMEM((128, TILE_L * 128), jnp.bfloat16),
            pltpu.VMEM((TILE_L * HID_W, B), jnp.float32),
        ],
        compiler_params=pltpu.CompilerParams(
            dimension_semantics=("arbitrary",),
        ),
    )(x2, v, node_weights, nb_row, w1d, b1t, w2s, b2s)
    return out.reshape(*orig_shape[:-1], OUT_W)
